# baseline (speedup 1.0000x reference)
import functools

import jax
import jax.numpy as jnp
from jax import lax
from jax.experimental import pallas as pl
from jax.experimental.pallas import tpu as pltpu
from jax.experimental.pallas import tpu_sc as plsc

NC = 2
NS = 16
NW = NC * NS
W = 128
NBUF = 4
TUNROLL = 8


@functools.lru_cache(maxsize=None)
def _make_lookup(S0, S1, D):
    B = S0 * S1
    NA = S0 // W
    b_per_w = B // NW
    nblk = b_per_w // W
    ngroup = nblk // NBUF
    assert b_per_w * NW == B and W * nblk == b_per_w and NBUF * ngroup == nblk
    assert D == 64 and S0 % W == 0

    mesh = plsc.VectorSubcoreMesh(core_axis_name="c", subcore_axis_name="s")

    @functools.partial(
        pl.kernel,
        mesh=mesh,
        compiler_params=pltpu.CompilerParams(
            use_tc_tiling_on_sc=False, needs_layout_passes=False),
        out_type=(
            jax.ShapeDtypeStruct((S1, 8, NA, 8, W), jnp.float32),
            jax.ShapeDtypeStruct((S1, 8, NA, 8, W), jnp.float32),
        ),
        scratch_types=(
            [pltpu.VMEM((b_per_w,), jnp.int32)] * 2
            + [pltpu.VMEM((W, D), jnp.float32)] * NBUF
            + [pltpu.VMEM((8, 8, W + 1), jnp.float32)] * NBUF
            + [pltpu.SemaphoreType.DMA] * (2 * NBUF)
        ),
    )
    def lookup(tok_idx_hbm, pos_idx_hbm, tok_tab, pos_tab, tok_out, pos_out,
               tok_idx_v, pos_idx_v, *scratch):
        rows = scratch[:NBUF]
        rowsT = scratch[NBUF:2 * NBUF]
        gsems = scratch[2 * NBUF:3 * NBUF]
        osems = scratch[3 * NBUF:]

        wid = lax.axis_index("s") * NC + lax.axis_index("c")
        ibase = pl.multiple_of(wid * b_per_w, 8)
        gbase = wid * nblk

        pltpu.sync_copy(tok_idx_hbm.at[pl.ds(ibase, b_per_w)], tok_idx_v)
        pltpu.sync_copy(pos_idx_hbm.at[pl.ds(ibase, b_per_w)], pos_idx_v)

        lane = lax.broadcasted_iota(jnp.int32, (16,), 0)
        dt_vecs = [(seg * 16 + lane) // 8 for seg in range(D // 16)]
        dl_vecs = [lane % 8] * (D // 16)

        def run_table(tab, idx_v, out):
            def gdesc(k, b):
                start = pl.multiple_of(k * W, 8)
                return pltpu.make_async_copy(
                    tab.at[idx_v.at[pl.ds(start, W)]], rows[b], gsems[b])

            def odesc(k, b):
                g = gbase + k
                bb = g // NA
                at = g % NA
                return pltpu.make_async_copy(
                    rowsT[b].at[:, :, pl.ds(0, W)], out.at[bb, :, at, :, :],
                    osems[b])

            def transpose(b):
                src = rows[b]
                dst = rowsT[b]

                def tbody(j, carry):
                    al0 = j * TUNROLL
                    for i in range(TUNROLL):
                        al16 = jnp.full((16,), al0 + i, dtype=jnp.int32)
                        vecs = [src[al0 + i, pl.ds(seg * 16, 16)]
                                for seg in range(D // 16)]
                        for seg in range(D // 16):
                            plsc.store_scatter(
                                dst, [dt_vecs[seg], dl_vecs[seg], al16],
                                vecs[seg])
                    return carry

                lax.fori_loop(0, W // TUNROLL, tbody, 0)

            for b in range(NBUF):
                gdesc(b, b).start()

            def body(j, carry):
                for b in range(NBUF):
                    k = j * NBUF + b
                    gdesc(k, b).wait()
                    transpose(b)
                    odesc(k, b).start()
                for b in range(NBUF):
                    k = j * NBUF + b
                    odesc(k, b).wait()

                    @pl.when(j < ngroup - 1)
                    def _():
                        gdesc(k + NBUF, b).start()
                return carry

            lax.fori_loop(0, ngroup, body, 0)

        run_table(tok_tab, tok_idx_v, tok_out)
        run_table(pos_tab, pos_idx_v, pos_out)

    return lookup


@functools.lru_cache(maxsize=None)
def _make_format(V, D):
    FULL = V // W
    REM = V - FULL * W
    FB = 3
    NBLK = (FULL - 1) // NW + 2
    NGRP = (NBLK + FB - 1) // FB
    mesh = plsc.VectorSubcoreMesh(core_axis_name="c", subcore_axis_name="s")

    @functools.partial(
        pl.kernel,
        mesh=mesh,
        compiler_params=pltpu.CompilerParams(
            use_tc_tiling_on_sc=True, needs_layout_passes=False),
        out_type=jax.ShapeDtypeStruct((V * D // W, W), jnp.float32),
        scratch_types=(
            [pltpu.VMEM((D, W + 5), jnp.float32)] * FB
            + [pltpu.VMEM((D, W), jnp.float32)] * FB
            + [pltpu.VMEM((REM * D // W, W), jnp.float32)]
            + [pltpu.SemaphoreType.DMA] * (2 * FB)
        ),
    )
    def fmt(tab_t, tail2, out, *scratch):
        bin_ = scratch[:FB]
        bout = scratch[FB:2 * FB]
        tbuf = scratch[2 * FB]
        isems = scratch[2 * FB + 1:3 * FB + 1]
        osems = scratch[3 * FB + 1:]

        wid = lax.axis_index("s") * NC + lax.axis_index("c")
        lane = lax.broadcasted_iota(jnp.int32, (16,), 0)
        dvecs = [seg * 16 + lane for seg in range(D // 16)]

        @pl.when(wid == 0)
        def _():
            pltpu.sync_copy(tail2, tbuf)
            pltpu.sync_copy(tbuf, out.at[pl.ds(FULL * D, REM * D // W), :])

        def cof(t):
            return wid + NW * t

        def idesc(t, b):
            c = cof(t)
            return pltpu.make_async_copy(
                tab_t.at[:, pl.ds(pl.multiple_of(c * W, 8), W)],
                bin_[b].at[:, pl.ds(0, W)], isems[b])

        def odesc(t, b):
            c = cof(t)
            return pltpu.make_async_copy(
                bout[b],
                out.at[pl.ds(pl.multiple_of(c * D, 8), D), :],
                osems[b])

        def istart(t, b):
            @pl.when(cof(t) < FULL)
            def _():
                idesc(t, b).start()

        def iwait(t, b):
            @pl.when(cof(t) < FULL)
            def _():
                idesc(t, b).wait()

        def ostart(t, b):
            @pl.when(cof(t) < FULL)
            def _():
                odesc(t, b).start()

        def owait(t, b):
            @pl.when(cof(t) < FULL)
            def _():
                odesc(t, b).wait()

        def xpose(b):
            src = bin_[b]
            dst = bout[b]

            def jbody(j, carry):
                for half in range(2):
                    il16 = jnp.full((16,), 2 * j + half, dtype=jnp.int32)
                    vecs = [plsc.load_gather(src, [dvecs[seg], il16])
                            for seg in range(D // 16)]
                    for seg in range(D // 16):
                        dst[j, pl.ds(half * D + seg * 16, 16)] = vecs[seg]
                return carry

            lax.fori_loop(0, D, jbody, 0)

        for b in range(FB):
            istart(b, b)

        def body(g, carry):
            for b in range(FB):
                t = g * FB + b
                iwait(t, b)
                xpose(b)
                ostart(t, b)
            for b in range(FB):
                t = g * FB + b
                owait(t, b)
                istart(t + FB, b)
            return carry

        lax.fori_loop(0, NGRP, body, 0)

    return fmt


def kernel(tokens, pos, token_table, pos_table):
    S0, S1 = tokens.shape
    B = S0 * S1
    D = token_table.shape[1]
    tok_flat = tokens.T.reshape(B).astype(jnp.int32)
    pos_flat = pos.T.reshape(B).astype(jnp.int32)
    V = token_table.shape[0]
    full = (V // 128) * 128
    tail2 = token_table[full:].reshape((V - full) * D // 128, 128)
    tableC = _make_format(V, D)(token_table.T, tail2)
    tableL = tableC.reshape(V, D)
    tok5, pos5 = _make_lookup(S0, S1, D)(
        tok_flat, pos_flat, tableL, pos_table)

    def to_entry(o5):
        return o5.transpose(2, 4, 0, 1, 3).reshape(S0, S1, D)

    return to_entry(tok5), to_entry(pos5)


# speedup vs baseline: 1.5673x; 1.5673x over previous
import functools

import jax
import jax.numpy as jnp
from jax import lax
from jax.experimental import pallas as pl
from jax.experimental.pallas import tpu as pltpu
from jax.experimental.pallas import tpu_sc as plsc

NC = 2
NS = 16
NW = NC * NS
W = 128
NBUF = 4
TUNROLL = 8


@functools.lru_cache(maxsize=None)
def _make_lookup(S0, S1, D):
    B = S0 * S1
    NA = S0 // W
    b_per_w = B // NW
    nblk = b_per_w // W
    ngroup = nblk // NBUF
    assert b_per_w * NW == B and W * nblk == b_per_w and NBUF * ngroup == nblk
    assert D == 64 and S0 % W == 0

    mesh = plsc.VectorSubcoreMesh(core_axis_name="c", subcore_axis_name="s")

    @functools.partial(
        pl.kernel,
        mesh=mesh,
        compiler_params=pltpu.CompilerParams(
            use_tc_tiling_on_sc=False, needs_layout_passes=False),
        out_type=(
            jax.ShapeDtypeStruct((S1, 8, NA, 8, W), jnp.float32),
            jax.ShapeDtypeStruct((S1, 8, NA, 8, W), jnp.float32),
        ),
        scratch_types=(
            [pltpu.VMEM((b_per_w,), jnp.int32)] * 2
            + [pltpu.VMEM((W, D), jnp.float32)] * NBUF
            + [pltpu.VMEM((8, 8, W + 1), jnp.float32)] * NBUF
            + [pltpu.SemaphoreType.DMA] * (2 * NBUF)
        ),
    )
    def lookup(tok_idx_hbm, pos_idx_hbm, tok_tab, pos_tab, tok_out, pos_out,
               tok_idx_v, pos_idx_v, *scratch):
        rows = scratch[:NBUF]
        rowsT = scratch[NBUF:2 * NBUF]
        gsems = scratch[2 * NBUF:3 * NBUF]
        osems = scratch[3 * NBUF:]

        wid = lax.axis_index("s") * NC + lax.axis_index("c")
        ibase = pl.multiple_of(wid * b_per_w, 8)
        gbase = wid * nblk

        pltpu.sync_copy(tok_idx_hbm.at[pl.ds(ibase, b_per_w)], tok_idx_v)
        pltpu.sync_copy(pos_idx_hbm.at[pl.ds(ibase, b_per_w)], pos_idx_v)

        lane = lax.broadcasted_iota(jnp.int32, (16,), 0)
        dt_vecs = [(seg * 16 + lane) // 8 for seg in range(D // 16)]
        dl_vecs = [lane % 8] * (D // 16)

        def run_table(tab, idx_v, out):
            def gdesc(k, b):
                start = pl.multiple_of(k * W, 8)
                return pltpu.make_async_copy(
                    tab.at[idx_v.at[pl.ds(start, W)]], rows[b], gsems[b])

            def odesc(k, b):
                g = gbase + k
                bb = g // NA
                at = g % NA
                return pltpu.make_async_copy(
                    rowsT[b].at[:, :, pl.ds(0, W)], out.at[bb, :, at, :, :],
                    osems[b])

            def transpose(b):
                src = rows[b]
                dst = rowsT[b]

                def tbody(j, carry):
                    al0 = j * TUNROLL
                    for i in range(TUNROLL):
                        al16 = jnp.full((16,), al0 + i, dtype=jnp.int32)
                        vecs = [src[al0 + i, pl.ds(seg * 16, 16)]
                                for seg in range(D // 16)]
                        for seg in range(D // 16):
                            plsc.store_scatter(
                                dst, [dt_vecs[seg], dl_vecs[seg], al16],
                                vecs[seg])
                    return carry

                lax.fori_loop(0, W // TUNROLL, tbody, 0)

            for b in range(NBUF):
                gdesc(b, b).start()

            def body(j, carry):
                for b in range(NBUF):
                    k = j * NBUF + b
                    gdesc(k, b).wait()
                    transpose(b)
                    odesc(k, b).start()
                for b in range(NBUF):
                    k = j * NBUF + b
                    odesc(k, b).wait()

                    @pl.when(j < ngroup - 1)
                    def _():
                        gdesc(k + NBUF, b).start()
                return carry

            lax.fori_loop(0, ngroup, body, 0)

        run_table(tok_tab, tok_idx_v, tok_out)
        run_table(pos_tab, pos_idx_v, pos_out)

    return lookup


def kernel(tokens, pos, token_table, pos_table):
    S0, S1 = tokens.shape
    B = S0 * S1
    D = token_table.shape[1]
    tok_flat = tokens.T.reshape(B).astype(jnp.int32)
    pos_flat = pos.T.reshape(B).astype(jnp.int32)
    tok5, pos5 = _make_lookup(S0, S1, D)(
        tok_flat, pos_flat, token_table, pos_table)

    def to_entry(o5):
        return o5.transpose(2, 4, 0, 1, 3).reshape(S0, S1, D)

    return to_entry(tok5), to_entry(pos5)
